# baseline (speedup 1.0000x reference)
import jax
import jax.numpy as jnp
from jax import lax
from jax.experimental import pallas as pl
from jax.experimental.pallas import tpu as pltpu
from jax.experimental.pallas import tpu_sc as plsc

N_ATOMS = 6400000
NUM_GRAPHS = 100000
LANES = 128
E_ROWS = N_ATOMS // LANES

CHUNKS = 4
TC_BLOCK_ROWS = 400
TC_BLOCK_ATOMS = TC_BLOCK_ROWS * LANES
CHUNK_BLOCKS = (8, 38, 39, 40)
CHUNK_ROWS = tuple(b * TC_BLOCK_ROWS for b in CHUNK_BLOCKS)
CHUNK_BASE_ROW = (0, 3200, 18400, 34000)

NUM_CORES = 2
NUM_SUBCORES = 16
NUM_WORKERS = NUM_CORES * NUM_SUBCORES
SC_PARAMS = (
    (100, 1, 100, 0),
    (475, 5, 95, 0),
    (485, 5, 97, 80),
    (500, 5, 100, 0),
)
ACC_PER_TILE = 6272
ACC_PAD = NUM_SUBCORES * ACC_PER_TILE


def _forces_energy_body(*refs):
    if len(refs) == 5:
        p_ref, p0_ref, _, f_ref, e_ref = refs
    else:
        p_ref, p0_ref, f_ref, e_ref = refs
    p = p_ref[...]
    p0 = p0_ref[...]
    dr = p - p0
    f_ref[...] = p0 - p
    sq = dr * dr
    e_row = (sq[0] + sq[1] + sq[2]) * jnp.float32(0.5)
    e_ref[...] = e_row.reshape(TC_BLOCK_ROWS, LANES)


def _make_forces_energy(chunk, first):
    base_blk = CHUNK_BASE_ROW[chunk] // TC_BLOCK_ROWS
    blk = lambda i, b=base_blk: (0, b + i)
    in_specs = [
        pl.BlockSpec((3, TC_BLOCK_ATOMS), blk),
        pl.BlockSpec((3, TC_BLOCK_ATOMS), blk),
    ]
    if not first:
        in_specs.append(pl.BlockSpec(memory_space=pl.ANY))
    return pl.pallas_call(
        _forces_energy_body,
        grid=(CHUNK_BLOCKS[chunk],),
        in_specs=in_specs,
        out_specs=[
            pl.BlockSpec((3, TC_BLOCK_ATOMS), blk),
            pl.BlockSpec((TC_BLOCK_ROWS, LANES), lambda i: (i, 0)),
        ],
        out_shape=[
            jax.ShapeDtypeStruct((3, N_ATOMS), jnp.float32),
            jax.ShapeDtypeStruct((CHUNK_ROWS[chunk], LANES), jnp.float32),
        ],
        input_output_aliases={} if first else {2: 0},
    )


_forces_energy = [_make_forces_energy(c, c == 0) for c in range(CHUNKS)]


def _make_segment_sum(chunk):
    chunk_base = CHUNK_BASE_ROW[chunk] * LANES
    rows_pw, n_stages, stage_rows, tail_rows = SC_PARAMS[chunk]
    stage_atoms = stage_rows * LANES
    tail_atoms = max(tail_rows, 16) * LANES

    def body(e_hbm, b_hbm, out_hbm, accum,
             e_buf0, i_buf0, e_buf1, i_buf1, te_buf, ti_buf, zbuf,
             sl0, sl1, ss0, ss1):
        c = lax.axis_index("c")
        s = lax.axis_index("s")
        w = s * NUM_CORES + c
        e_bufs = (e_buf0, e_buf1)
        i_bufs = (i_buf0, i_buf1)
        sem_l = (sl0, sl1)
        sem_s = (ss0, ss1)

        def _zero(j, _):
            zbuf[pl.ds(j * 16, 16)] = jnp.zeros((16,), jnp.float32)
            return 0
        lax.fori_loop(0, ACC_PER_TILE // 16, _zero, 0, unroll=8)
        pltpu.sync_copy(zbuf, accum.at[pl.ds(s * ACC_PER_TILE, ACC_PER_TILE)])
        plsc.subcore_barrier()

        base = w * rows_pw * LANES

        def _start_load(k):
            b = k % 2
            a0 = base + k * stage_atoms
            dl_e = pltpu.async_copy(e_hbm.at[pl.ds(a0, stage_atoms)],
                                    e_bufs[b], sem_l[b])
            dl_i = pltpu.async_copy(
                b_hbm.at[pl.ds(chunk_base + a0, stage_atoms)],
                i_bufs[b], sem_l[b])
            return dl_e, dl_i

        loads = _start_load(0)
        scatters = [None, None]
        for k in range(n_stages):
            b = k % 2
            loads[0].wait()
            loads[1].wait()
            if k + 1 < n_stages:
                if scatters[(k + 1) % 2] is not None:
                    scatters[(k + 1) % 2].wait()
                    scatters[(k + 1) % 2] = None
                loads = _start_load(k + 1)
            scatters[b] = pltpu.async_copy(e_bufs[b], accum.at[i_bufs[b]],
                                           sem_s[b], add=True)
        for b in range(2):
            if scatters[b] is not None:
                scatters[b].wait()

        if tail_rows:
            @pl.when(w == 0)
            def _tail():
                a0 = NUM_WORKERS * rows_pw * LANES
                pltpu.sync_copy(e_hbm.at[pl.ds(a0, tail_atoms)], te_buf)
                pltpu.sync_copy(b_hbm.at[pl.ds(chunk_base + a0, tail_atoms)],
                                ti_buf)
                pltpu.sync_copy(te_buf, accum.at[ti_buf], add=True)

        plsc.subcore_barrier()
        pltpu.sync_copy(accum.at[pl.ds(s * ACC_PER_TILE, ACC_PER_TILE)],
                        out_hbm.at[c, s])

    return pl.kernel(
        body,
        out_type=jax.ShapeDtypeStruct((NUM_CORES, NUM_SUBCORES, ACC_PER_TILE),
                                      jnp.float32),
        mesh=plsc.VectorSubcoreMesh(core_axis_name="c", subcore_axis_name="s"),
        scratch_types=[
            pltpu.VMEM_SHARED((ACC_PAD,), jnp.float32),
            pltpu.VMEM((stage_atoms,), jnp.float32),
            pltpu.VMEM((stage_atoms,), jnp.int32),
            pltpu.VMEM((stage_atoms,), jnp.float32),
            pltpu.VMEM((stage_atoms,), jnp.int32),
            pltpu.VMEM((tail_atoms,), jnp.float32),
            pltpu.VMEM((tail_atoms,), jnp.int32),
            pltpu.VMEM((ACC_PER_TILE,), jnp.float32),
            pltpu.SemaphoreType.DMA,
            pltpu.SemaphoreType.DMA,
            pltpu.SemaphoreType.DMA,
            pltpu.SemaphoreType.DMA,
        ],
    )


_segment_sum = [_make_segment_sum(c) for c in range(CHUNKS)]


def _combine_body(p0, p1, p2, p3, o_ref):
    acc = None
    for p in (p0, p1, p2, p3):
        for i in range(NUM_CORES):
            acc = p[i] if acc is None else acc + p[i]
    o_ref[...] = acc


_combine = pl.pallas_call(
    _combine_body,
    in_specs=[pl.BlockSpec((NUM_CORES, ACC_PAD // LANES, LANES),
                           lambda: (0, 0, 0))] * CHUNKS,
    out_specs=pl.BlockSpec((ACC_PAD // LANES, LANES), lambda: (0, 0)),
    out_shape=jax.ShapeDtypeStruct((ACC_PAD // LANES, LANES), jnp.float32),
)


@jax.jit
def kernel(pos, pos0, batch):
    pos_t = pos.T
    pos0_t = pos0.T
    forces_t = None
    partials = []
    for c in range(CHUNKS):
        if c == 0:
            forces_t, e2 = _forces_energy[c](pos_t, pos0_t)
        else:
            forces_t, e2 = _forces_energy[c](pos_t, pos0_t, forces_t)
        partials.append(
            _segment_sum[c](e2.reshape(CHUNK_ROWS[c] * LANES), batch))
    combined = _combine(
        *[p.reshape(NUM_CORES, ACC_PAD // LANES, LANES) for p in partials])
    energy = combined.reshape(ACC_PAD)[:NUM_GRAPHS]
    return energy, forces_t.T


# speedup vs baseline: 1.0679x; 1.0679x over previous
import jax
import jax.numpy as jnp
from jax import lax
from jax.experimental import pallas as pl
from jax.experimental.pallas import tpu as pltpu
from jax.experimental.pallas import tpu_sc as plsc

N_ATOMS = 6400000
NUM_GRAPHS = 100000
LANES = 128
E_ROWS = N_ATOMS // LANES

CHUNKS = 4
TC_BLOCK_ROWS = 400
TC_BLOCK_ATOMS = TC_BLOCK_ROWS * LANES
CHUNK_BLOCKS = (16, 36, 36, 37)
CHUNK_ROWS = tuple(b * TC_BLOCK_ROWS for b in CHUNK_BLOCKS)
CHUNK_BASE_ROW = (0, 6400, 20800, 35200)

NUM_CORES = 2
NUM_SUBCORES = 16
NUM_WORKERS = NUM_CORES * NUM_SUBCORES
SC_PARAMS = (
    (200, 2, 100, 0),
    (450, 3, 150, 0),
    (450, 3, 150, 0),
    (462, 3, 154, 16),
)
ACC_PER_TILE = 6272
ACC_PAD = NUM_SUBCORES * ACC_PER_TILE


def _forces_energy_body(*refs):
    if len(refs) == 5:
        p_ref, p0_ref, _, f_ref, e_ref = refs
    else:
        p_ref, p0_ref, f_ref, e_ref = refs
    p = p_ref[...]
    p0 = p0_ref[...]
    dr = p - p0
    f_ref[...] = p0 - p
    sq = dr * dr
    e_row = (sq[0] + sq[1] + sq[2]) * jnp.float32(0.5)
    e_ref[...] = e_row.reshape(TC_BLOCK_ROWS, LANES)


def _make_forces_energy(chunk, first):
    base_blk = CHUNK_BASE_ROW[chunk] // TC_BLOCK_ROWS
    blk = lambda i, b=base_blk: (0, b + i)
    in_specs = [
        pl.BlockSpec((3, TC_BLOCK_ATOMS), blk),
        pl.BlockSpec((3, TC_BLOCK_ATOMS), blk),
    ]
    if not first:
        in_specs.append(pl.BlockSpec(memory_space=pl.ANY))
    return pl.pallas_call(
        _forces_energy_body,
        grid=(CHUNK_BLOCKS[chunk],),
        in_specs=in_specs,
        out_specs=[
            pl.BlockSpec((3, TC_BLOCK_ATOMS), blk),
            pl.BlockSpec((TC_BLOCK_ROWS, LANES), lambda i: (i, 0)),
        ],
        out_shape=[
            jax.ShapeDtypeStruct((3, N_ATOMS), jnp.float32),
            jax.ShapeDtypeStruct((CHUNK_ROWS[chunk], LANES), jnp.float32),
        ],
        input_output_aliases={} if first else {2: 0},
    )


_forces_energy = [_make_forces_energy(c, c == 0) for c in range(CHUNKS)]


def _make_segment_sum(chunk):
    chunk_base = CHUNK_BASE_ROW[chunk] * LANES
    rows_pw, n_stages, stage_rows, tail_rows = SC_PARAMS[chunk]
    stage_atoms = stage_rows * LANES
    tail_atoms = max(tail_rows, 16) * LANES

    def body(e_hbm, b_hbm, out_hbm, accum,
             e_buf0, i_buf0, e_buf1, i_buf1, te_buf, ti_buf, zbuf,
             sl0, sl1, ss0, ss1):
        c = lax.axis_index("c")
        s = lax.axis_index("s")
        w = s * NUM_CORES + c
        e_bufs = (e_buf0, e_buf1)
        i_bufs = (i_buf0, i_buf1)
        sem_l = (sl0, sl1)
        sem_s = (ss0, ss1)

        def _zero(j, _):
            zbuf[pl.ds(j * 16, 16)] = jnp.zeros((16,), jnp.float32)
            return 0
        lax.fori_loop(0, ACC_PER_TILE // 16, _zero, 0, unroll=8)
        pltpu.sync_copy(zbuf, accum.at[pl.ds(s * ACC_PER_TILE, ACC_PER_TILE)])
        plsc.subcore_barrier()

        base = w * rows_pw * LANES

        def _start_load(k):
            b = k % 2
            a0 = base + k * stage_atoms
            dl_e = pltpu.async_copy(e_hbm.at[pl.ds(a0, stage_atoms)],
                                    e_bufs[b], sem_l[b])
            dl_i = pltpu.async_copy(
                b_hbm.at[pl.ds(chunk_base + a0, stage_atoms)],
                i_bufs[b], sem_l[b])
            return dl_e, dl_i

        loads = _start_load(0)
        scatters = [None, None]
        for k in range(n_stages):
            b = k % 2
            loads[0].wait()
            loads[1].wait()
            if k + 1 < n_stages:
                if scatters[(k + 1) % 2] is not None:
                    scatters[(k + 1) % 2].wait()
                    scatters[(k + 1) % 2] = None
                loads = _start_load(k + 1)
            scatters[b] = pltpu.async_copy(e_bufs[b], accum.at[i_bufs[b]],
                                           sem_s[b], add=True)
        for b in range(2):
            if scatters[b] is not None:
                scatters[b].wait()

        if tail_rows:
            @pl.when(w == 0)
            def _tail():
                a0 = NUM_WORKERS * rows_pw * LANES
                pltpu.sync_copy(e_hbm.at[pl.ds(a0, tail_atoms)], te_buf)
                pltpu.sync_copy(b_hbm.at[pl.ds(chunk_base + a0, tail_atoms)],
                                ti_buf)
                pltpu.sync_copy(te_buf, accum.at[ti_buf], add=True)

        plsc.subcore_barrier()
        pltpu.sync_copy(accum.at[pl.ds(s * ACC_PER_TILE, ACC_PER_TILE)],
                        out_hbm.at[c, s])

    return pl.kernel(
        body,
        out_type=jax.ShapeDtypeStruct((NUM_CORES, NUM_SUBCORES, ACC_PER_TILE),
                                      jnp.float32),
        mesh=plsc.VectorSubcoreMesh(core_axis_name="c", subcore_axis_name="s"),
        scratch_types=[
            pltpu.VMEM_SHARED((ACC_PAD,), jnp.float32),
            pltpu.VMEM((stage_atoms,), jnp.float32),
            pltpu.VMEM((stage_atoms,), jnp.int32),
            pltpu.VMEM((stage_atoms,), jnp.float32),
            pltpu.VMEM((stage_atoms,), jnp.int32),
            pltpu.VMEM((tail_atoms,), jnp.float32),
            pltpu.VMEM((tail_atoms,), jnp.int32),
            pltpu.VMEM((ACC_PER_TILE,), jnp.float32),
            pltpu.SemaphoreType.DMA,
            pltpu.SemaphoreType.DMA,
            pltpu.SemaphoreType.DMA,
            pltpu.SemaphoreType.DMA,
        ],
    )


_segment_sum = [_make_segment_sum(c) for c in range(CHUNKS)]


def _combine_body(p0, p1, p2, p3, o_ref):
    acc = None
    for p in (p0, p1, p2, p3):
        for i in range(NUM_CORES):
            acc = p[i] if acc is None else acc + p[i]
    o_ref[...] = acc


_combine = pl.pallas_call(
    _combine_body,
    in_specs=[pl.BlockSpec((NUM_CORES, ACC_PAD // LANES, LANES),
                           lambda: (0, 0, 0))] * CHUNKS,
    out_specs=pl.BlockSpec((ACC_PAD // LANES, LANES), lambda: (0, 0)),
    out_shape=jax.ShapeDtypeStruct((ACC_PAD // LANES, LANES), jnp.float32),
)


@jax.jit
def kernel(pos, pos0, batch):
    pos_t = pos.T
    pos0_t = pos0.T
    forces_t = None
    partials = []
    for c in range(CHUNKS):
        if c == 0:
            forces_t, e2 = _forces_energy[c](pos_t, pos0_t)
        else:
            forces_t, e2 = _forces_energy[c](pos_t, pos0_t, forces_t)
        partials.append(
            _segment_sum[c](e2.reshape(CHUNK_ROWS[c] * LANES), batch))
    combined = _combine(
        *[p.reshape(NUM_CORES, ACC_PAD // LANES, LANES) for p in partials])
    energy = combined.reshape(ACC_PAD)[:NUM_GRAPHS]
    return energy, forces_t.T
